# baseline (speedup 1.0000x reference)
import functools

import jax
import jax.numpy as jnp
from jax import lax
from jax.experimental import pallas as pl
from jax.experimental.pallas import tpu as pltpu
from jax.experimental.pallas import tpu_sc as plsc

E = 8
TOPK = 2
NEG_INF = -1e30

NC = 2
NS = 16
NW = NC * NS
CHUNK = 32

TR = 128


def _router_kernel(x_ref, rw_ref, xb_ref, wa_ref, wb_ref, pa_ref, pb_ref,
                   off_ref):
    x = x_ref[...]
    T = x.shape[0]
    xb = x.astype(jnp.bfloat16)
    xb_ref[...] = xb
    logits = jax.lax.dot_general(
        xb, rw_ref[...].astype(jnp.bfloat16), (((1,), (1,)), ((), ())),
        preferred_element_type=jnp.float32)
    lane8 = lax.broadcasted_iota(jnp.int32, (T, E), 1)
    m1 = jnp.max(logits, axis=1, keepdims=True)
    a1 = jnp.argmax(logits, axis=1, keepdims=True)
    masked = jnp.where(lane8 == a1, NEG_INF, logits)
    m2 = jnp.max(masked, axis=1, keepdims=True)
    a2 = jnp.argmax(masked, axis=1, keepdims=True)
    ex = jnp.exp(logits - m1)
    denom = jnp.sum(ex, axis=1, keepdims=True)
    wa_ref[...] = 1.0 / denom
    wb_ref[...] = jnp.exp(m2 - m1) / denom

    na = (lane8 == a1).astype(jnp.bfloat16)
    nb = (lane8 == a2).astype(jnp.bfloat16)
    n = na + nb
    rowi = lax.broadcasted_iota(jnp.int32, (T, T), 0)
    coli = lax.broadcasted_iota(jnp.int32, (T, T), 1)
    tri = jnp.where(coli < rowi, 1.0, 0.0).astype(jnp.bfloat16)
    cx = jax.lax.dot_general(
        tri, n, (((1,), (0,)), ((), ())),
        preferred_element_type=jnp.float32)
    tot = jnp.sum(n.astype(jnp.float32), axis=0, keepdims=True)
    e8r = lax.broadcasted_iota(jnp.int32, (E, E), 0)
    e8c = lax.broadcasted_iota(jnp.int32, (E, E), 1)
    tri8 = jnp.where(e8r < e8c, 1.0, 0.0)
    off = jax.lax.dot_general(
        tot, tri8, (((1,), (0,)), ((), ())),
        preferred_element_type=jnp.float32,
        precision=jax.lax.Precision.HIGHEST)
    dest = cx + off
    pa = jnp.sum(jnp.where(lane8 == a1, dest, 0.0), axis=1, keepdims=True)
    pb = jnp.sum(jnp.where(lane8 == a2, dest, 0.0), axis=1, keepdims=True)
    pa_ref[...] = pa.astype(jnp.int32)
    pb_ref[...] = pb.astype(jnp.int32)
    off_ref[...] = off.astype(jnp.int32)


def _sc_dispatch(xw, pall3):
    T, W = xw.shape
    S = TOPK * T
    per_w = S // NW
    n_chunks = per_w // CHUNK
    mesh = plsc.VectorSubcoreMesh(core_axis_name="c", subcore_axis_name="s")

    @functools.partial(
        pl.kernel, mesh=mesh,
        out_type=jax.ShapeDtypeStruct((S, W), jnp.int32),
        scratch_types=[
            pltpu.VMEM((n_chunks, CHUNK), jnp.int32),
            pltpu.VMEM((CHUNK, W), jnp.int32),
            pltpu.SemaphoreType.DMA,
        ],
    )
    def k(xw_hbm, idx_hbm, xs_hbm, idx_v, rows_v, sem):
        wid = lax.axis_index("s") * NC + lax.axis_index("c")
        pltpu.sync_copy(idx_hbm.at[wid], idx_v)
        tok_base = (wid % NS) * per_w
        for c in range(n_chunks):
            pltpu.sync_copy(xw_hbm.at[pl.ds(tok_base + c * CHUNK, CHUNK)],
                            rows_v)
            pltpu.async_copy(rows_v, xs_hbm.at[idx_v.at[c]], sem).wait()

    return k(xw, pall3)


def _sc_combine(yw, pall3):
    S, W = yw.shape
    per_w = S // NW
    n_chunks = per_w // CHUNK
    mesh = plsc.VectorSubcoreMesh(core_axis_name="c", subcore_axis_name="s")

    @functools.partial(
        pl.kernel, mesh=mesh,
        out_type=jax.ShapeDtypeStruct((S, W), jnp.int32),
        scratch_types=[
            pltpu.VMEM((n_chunks, CHUNK), jnp.int32),
            pltpu.VMEM((CHUNK, W), jnp.int32),
            pltpu.SemaphoreType.DMA,
        ],
    )
    def k(yw_hbm, idx_hbm, g_hbm, idx_v, rows_v, sem):
        wid = lax.axis_index("s") * NC + lax.axis_index("c")
        pltpu.sync_copy(idx_hbm.at[wid], idx_v)
        base = wid * per_w
        for c in range(n_chunks):
            pltpu.async_copy(yw_hbm.at[idx_v.at[c]], rows_v, sem).wait()
            pltpu.sync_copy(rows_v, g_hbm.at[pl.ds(base + c * CHUNK, CHUNK)])

    return k(yw, pall3)


def _gmm_kernel(tid_ref, gid_ref, lo_ref, hi_ref, xs_ref, w1_ref, w2_ref,
                y_ref):
    s = pl.program_id(0)
    lo = lo_ref[s]
    hi = hi_ref[s]

    @pl.when(lo < hi)
    def _():
        xs = xs_ref[...]
        h = jax.lax.dot_general(
            xs, w1_ref[...], (((1,), (1,)), ((), ())),
            preferred_element_type=jnp.float32)
        h = jnp.maximum(h, 0.0)
        h = h * h
        y = jax.lax.dot_general(
            h.astype(jnp.bfloat16), w2_ref[...], (((1,), (1,)), ((), ())),
            preferred_element_type=jnp.float32)
        rows = tid_ref[s] * TR + lax.broadcasted_iota(jnp.int32,
                                                      (TR, 1), 0)
        m = (rows >= lo) & (rows < hi)
        y_ref[...] = jnp.where(m, y.astype(jnp.bfloat16), y_ref[...])


def _final_kernel(xb_ref, ws1_ref, ws2_ref, wa_ref, wb_ref, ga_ref, gb_ref,
                  o_ref):
    xb = xb_ref[...]
    h = jax.lax.dot_general(
        xb, ws1_ref[...], (((1,), (1,)), ((), ())),
        preferred_element_type=jnp.float32)
    h = jnp.maximum(h, 0.0)
    h = h * h
    sh = jax.lax.dot_general(
        h.astype(jnp.bfloat16), ws2_ref[...], (((1,), (1,)), ((), ())),
        preferred_element_type=jnp.float32)
    o_ref[...] = (sh + wa_ref[...] * ga_ref[...].astype(jnp.float32)
                  + wb_ref[...] * gb_ref[...].astype(jnp.float32))


def kernel(hidden_tensor, router_w, w1_stack, w2_stack, shared_w1, shared_w2):
    B, T, C = hidden_tensor.shape
    F = w1_stack.shape[1]
    S = TOPK * T
    W = C // 2
    x = hidden_tensor.reshape(T, C)

    xb, wa, wb, pa, pb, off8 = pl.pallas_call(
        _router_kernel,
        out_shape=(
            jax.ShapeDtypeStruct((T, C), jnp.bfloat16),
            jax.ShapeDtypeStruct((T, 1), jnp.float32),
            jax.ShapeDtypeStruct((T, 1), jnp.float32),
            jax.ShapeDtypeStruct((T, 1), jnp.int32),
            jax.ShapeDtypeStruct((T, 1), jnp.int32),
            jax.ShapeDtypeStruct((1, E), jnp.int32),
        ),
    )(x, router_w)

    pall = jnp.concatenate([pa[:, 0], pb[:, 0]])
    pall3 = pall.reshape(NW, (S // NW) // CHUNK, CHUNK)

    xw = jax.lax.bitcast_convert_type(xb.reshape(T, W, 2), jnp.int32)
    xs_i32 = _sc_dispatch(xw, pall3)
    xs = jax.lax.bitcast_convert_type(xs_i32, jnp.bfloat16).reshape(S, C)

    NT = S // TR
    STEPS = NT + E
    off = jnp.concatenate([off8.reshape(E),
                           jnp.array([S], dtype=jnp.int32)])
    start, end = off[:-1], off[1:]
    size = end - start
    t_first = start // TR
    t_last = jnp.where(size > 0, (end - 1) // TR, t_first)
    num_t = jnp.where(size > 0, t_last - t_first + 1, 0)
    sbase = jnp.concatenate([jnp.zeros((1,), jnp.int32),
                             jnp.cumsum(num_t)]).astype(jnp.int32)
    total = sbase[E]
    sarr = jnp.arange(STEPS, dtype=jnp.int32)
    gos = jnp.minimum(
        jnp.sum((sbase[1:][None, :] <= sarr[:, None]).astype(jnp.int32),
                axis=1), E - 1)
    tid = t_first[gos] + (sarr - sbase[:-1][gos])
    valid = sarr < total
    lo = start[gos]
    hi = jnp.where(valid, end[gos], lo)
    tid = jnp.where(valid, tid, NT - 1).astype(jnp.int32)
    gid = gos.astype(jnp.int32)

    w1b = w1_stack.astype(jnp.bfloat16)
    w2b = w2_stack.astype(jnp.bfloat16)
    grid_spec = pltpu.PrefetchScalarGridSpec(
        num_scalar_prefetch=4,
        grid=(STEPS,),
        in_specs=[
            pl.BlockSpec((TR, C), lambda s, tid, gid, lo, hi: (tid[s], 0)),
            pl.BlockSpec((None, F, C),
                         lambda s, tid, gid, lo, hi: (gid[s], 0, 0)),
            pl.BlockSpec((None, C, F),
                         lambda s, tid, gid, lo, hi: (gid[s], 0, 0)),
        ],
        out_specs=pl.BlockSpec((TR, C),
                               lambda s, tid, gid, lo, hi: (tid[s], 0)),
    )
    y = pl.pallas_call(
        _gmm_kernel,
        grid_spec=grid_spec,
        out_shape=jax.ShapeDtypeStruct((S, C), jnp.bfloat16),
    )(tid, gid, lo, hi, xs, w1b, w2b)

    yw = jax.lax.bitcast_convert_type(y.reshape(S, W, 2), jnp.int32)
    g_i32 = _sc_combine(yw, pall3)
    gall = jax.lax.bitcast_convert_type(g_i32, jnp.bfloat16).reshape(S, C)

    TB = 1024
    NB = T // TB
    ws1 = shared_w1.astype(jnp.bfloat16)
    ws2 = shared_w2.astype(jnp.bfloat16)
    out = pl.pallas_call(
        _final_kernel,
        grid=(NB,),
        in_specs=[
            pl.BlockSpec((TB, C), lambda t: (t, 0)),
            pl.BlockSpec((F, C), lambda t: (0, 0)),
            pl.BlockSpec((C, F), lambda t: (0, 0)),
            pl.BlockSpec((TB, 1), lambda t: (t, 0)),
            pl.BlockSpec((TB, 1), lambda t: (t, 0)),
            pl.BlockSpec((TB, C), lambda t: (t, 0)),
            pl.BlockSpec((TB, C), lambda t: (t + NB, 0)),
        ],
        out_specs=pl.BlockSpec((TB, C), lambda t: (t, 0)),
        out_shape=jax.ShapeDtypeStruct((T, C), jnp.float32),
    )(xb, ws1, ws2, wa, wb, gall, gall)

    return out.reshape(B, T, C)


# speedup vs baseline: 2.3658x; 2.3658x over previous
import functools

import jax
import jax.numpy as jnp
from jax import lax
from jax.experimental import pallas as pl
from jax.experimental.pallas import tpu as pltpu
from jax.experimental.pallas import tpu_sc as plsc

E = 8
TOPK = 2
NEG_INF = -1e30

NC = 2
NS = 16
NW = NC * NS
CHUNK = 32

TR = 128


def _router_kernel(x_ref, rw_ref, xb_ref, wa_ref, wb_ref, pa_ref, pb_ref,
                   off_ref):
    x = x_ref[...]
    T = x.shape[0]
    xb_ref[...] = x.reshape(T, 16, 128)
    logits = jax.lax.dot_general(
        x.astype(jnp.bfloat16), rw_ref[...].astype(jnp.bfloat16),
        (((1,), (1,)), ((), ())),
        preferred_element_type=jnp.float32)
    lane8 = lax.broadcasted_iota(jnp.int32, (T, E), 1)
    m1 = jnp.max(logits, axis=1, keepdims=True)
    a1 = jnp.argmax(logits, axis=1, keepdims=True)
    masked = jnp.where(lane8 == a1, NEG_INF, logits)
    m2 = jnp.max(masked, axis=1, keepdims=True)
    a2 = jnp.argmax(masked, axis=1, keepdims=True)
    ex = jnp.exp(logits - m1)
    denom = jnp.sum(ex, axis=1, keepdims=True)
    wa_ref[...] = 1.0 / denom
    wb_ref[...] = jnp.exp(m2 - m1) / denom

    na = (lane8 == a1).astype(jnp.bfloat16)
    nb = (lane8 == a2).astype(jnp.bfloat16)
    n = na + nb
    rowi = lax.broadcasted_iota(jnp.int32, (T, T), 0)
    coli = lax.broadcasted_iota(jnp.int32, (T, T), 1)
    tri = jnp.where(coli < rowi, 1.0, 0.0).astype(jnp.bfloat16)
    cx = jax.lax.dot_general(
        tri, n, (((1,), (0,)), ((), ())),
        preferred_element_type=jnp.float32)
    tot = jnp.sum(n.astype(jnp.float32), axis=0, keepdims=True)
    e8r = lax.broadcasted_iota(jnp.int32, (E, E), 0)
    e8c = lax.broadcasted_iota(jnp.int32, (E, E), 1)
    tri8 = jnp.where(e8r < e8c, 1.0, 0.0)
    off = jax.lax.dot_general(
        tot, tri8, (((1,), (0,)), ((), ())),
        preferred_element_type=jnp.float32,
        precision=jax.lax.Precision.HIGHEST)
    dest = cx + off
    pa = jnp.sum(jnp.where(lane8 == a1, dest, 0.0), axis=1, keepdims=True)
    pb = jnp.sum(jnp.where(lane8 == a2, dest, 0.0), axis=1, keepdims=True)
    pa_ref[...] = pa.astype(jnp.int32)
    pb_ref[...] = pb.astype(jnp.int32)
    off_ref[...] = off.astype(jnp.int32)


def _sc_dispatch(xw, pall3):
    T = xw.shape[0]
    S = TOPK * T
    per_w = S // NW
    n_chunks = per_w // CHUNK
    mesh = plsc.VectorSubcoreMesh(core_axis_name="c", subcore_axis_name="s")

    @functools.partial(
        pl.kernel, mesh=mesh,
        out_type=jax.ShapeDtypeStruct((S, 16, 128), jnp.float32),
        scratch_types=[
            pltpu.VMEM((n_chunks, CHUNK), jnp.int32),
            pltpu.VMEM((CHUNK, 16, 128), jnp.float32),
            pltpu.SemaphoreType.DMA,
        ],
    )
    def k(xw_hbm, idx_hbm, xs_hbm, idx_v, rows_v, sem):
        wid = lax.axis_index("s") * NC + lax.axis_index("c")
        pltpu.sync_copy(idx_hbm.at[wid], idx_v)
        tok_base = (wid % NS) * per_w
        for c in range(n_chunks):
            pltpu.sync_copy(xw_hbm.at[pl.ds(tok_base + c * CHUNK, CHUNK)],
                            rows_v)
            pltpu.async_copy(rows_v, xs_hbm.at[idx_v.at[c]], sem).wait()

    return k(xw, pall3)


def _sc_combine(yw, pall3):
    S = yw.shape[0]
    per_w = S // NW
    n_chunks = per_w // CHUNK
    mesh = plsc.VectorSubcoreMesh(core_axis_name="c", subcore_axis_name="s")

    @functools.partial(
        pl.kernel, mesh=mesh,
        out_type=jax.ShapeDtypeStruct((S, 16, 128), jnp.float32),
        scratch_types=[
            pltpu.VMEM((n_chunks, CHUNK), jnp.int32),
            pltpu.VMEM((CHUNK, 16, 128), jnp.float32),
            pltpu.SemaphoreType.DMA,
        ],
    )
    def k(yw_hbm, idx_hbm, g_hbm, idx_v, rows_v, sem):
        wid = lax.axis_index("s") * NC + lax.axis_index("c")
        pltpu.sync_copy(idx_hbm.at[wid], idx_v)
        base = wid * per_w
        for c in range(n_chunks):
            pltpu.async_copy(yw_hbm.at[idx_v.at[c]], rows_v, sem).wait()
            pltpu.sync_copy(rows_v, g_hbm.at[pl.ds(base + c * CHUNK, CHUNK)])

    return k(yw, pall3)


def _gmm_kernel(tid_ref, gid_ref, lo_ref, hi_ref, xs_ref, w1_ref, w2_ref,
                y_ref):
    s = pl.program_id(0)
    lo = lo_ref[s]
    hi = hi_ref[s]

    @pl.when(lo < hi)
    def _():
        xs = xs_ref[...].reshape(TR, 2048).astype(jnp.bfloat16)
        h = jax.lax.dot_general(
            xs, w1_ref[...], (((1,), (1,)), ((), ())),
            preferred_element_type=jnp.float32)
        h = jnp.maximum(h, 0.0)
        h = h * h
        y = jax.lax.dot_general(
            h.astype(jnp.bfloat16), w2_ref[...], (((1,), (1,)), ((), ())),
            preferred_element_type=jnp.float32)
        rows = tid_ref[s] * TR + lax.broadcasted_iota(jnp.int32,
                                                      (TR, 1, 1), 0)
        m = (rows >= lo) & (rows < hi)
        y3 = y.reshape(TR, 16, 128)
        y_ref[...] = jnp.where(m, y3, y_ref[...])


def _final_kernel(xb_ref, ws1_ref, ws2_ref, wa_ref, wb_ref, ga_ref, gb_ref,
                  o_ref):
    xb3 = xb_ref[...]
    xb = xb3.reshape(xb3.shape[0], 2048).astype(jnp.bfloat16)
    h = jax.lax.dot_general(
        xb, ws1_ref[...], (((1,), (1,)), ((), ())),
        preferred_element_type=jnp.float32)
    h = jnp.maximum(h, 0.0)
    h = h * h
    sh = jax.lax.dot_general(
        h.astype(jnp.bfloat16), ws2_ref[...], (((1,), (1,)), ((), ())),
        preferred_element_type=jnp.float32)
    tb = sh.shape[0]
    ga = ga_ref[...].reshape(tb, 2048)
    gb = gb_ref[...].reshape(tb, 2048)
    o_ref[...] = sh + wa_ref[...] * ga + wb_ref[...] * gb


def kernel(hidden_tensor, router_w, w1_stack, w2_stack, shared_w1, shared_w2):
    B, T, C = hidden_tensor.shape
    F = w1_stack.shape[1]
    S = TOPK * T
    W = C // 2
    x = hidden_tensor.reshape(T, C)

    xb3, wa, wb, pa, pb, off8 = pl.pallas_call(
        _router_kernel,
        out_shape=(
            jax.ShapeDtypeStruct((T, 16, 128), jnp.float32),
            jax.ShapeDtypeStruct((T, 1), jnp.float32),
            jax.ShapeDtypeStruct((T, 1), jnp.float32),
            jax.ShapeDtypeStruct((T, 1), jnp.int32),
            jax.ShapeDtypeStruct((T, 1), jnp.int32),
            jax.ShapeDtypeStruct((1, E), jnp.int32),
        ),
    )(x, router_w)

    pall = jnp.concatenate([pa[:, 0], pb[:, 0]])
    pall3 = pall.reshape(NW, (S // NW) // CHUNK, CHUNK)

    xs3 = _sc_dispatch(xb3, pall3)

    NT = S // TR
    STEPS = NT + E
    off = jnp.concatenate([off8.reshape(E),
                           jnp.array([S], dtype=jnp.int32)])
    start, end = off[:-1], off[1:]
    size = end - start
    t_first = start // TR
    t_last = jnp.where(size > 0, (end - 1) // TR, t_first)
    num_t = jnp.where(size > 0, t_last - t_first + 1, 0)
    sbase = jnp.concatenate([jnp.zeros((1,), jnp.int32),
                             jnp.cumsum(num_t)]).astype(jnp.int32)
    total = sbase[E]
    sarr = jnp.arange(STEPS, dtype=jnp.int32)
    gos = jnp.minimum(
        jnp.sum((sbase[1:][None, :] <= sarr[:, None]).astype(jnp.int32),
                axis=1), E - 1)
    tid = t_first[gos] + (sarr - sbase[:-1][gos])
    valid = sarr < total
    lo = start[gos]
    hi = jnp.where(valid, end[gos], lo)
    tid = jnp.where(valid, tid, NT - 1).astype(jnp.int32)
    gid = gos.astype(jnp.int32)

    w1b = w1_stack.astype(jnp.bfloat16)
    w2b = w2_stack.astype(jnp.bfloat16)
    grid_spec = pltpu.PrefetchScalarGridSpec(
        num_scalar_prefetch=4,
        grid=(STEPS,),
        in_specs=[
            pl.BlockSpec((TR, 16, 128),
                         lambda s, tid, gid, lo, hi: (tid[s], 0, 0)),
            pl.BlockSpec((None, F, C),
                         lambda s, tid, gid, lo, hi: (gid[s], 0, 0)),
            pl.BlockSpec((None, C, F),
                         lambda s, tid, gid, lo, hi: (gid[s], 0, 0)),
        ],
        out_specs=pl.BlockSpec((TR, 16, 128),
                               lambda s, tid, gid, lo, hi: (tid[s], 0, 0)),
    )
    y3 = pl.pallas_call(
        _gmm_kernel,
        grid_spec=grid_spec,
        out_shape=jax.ShapeDtypeStruct((S, 16, 128), jnp.float32),
    )(tid, gid, lo, hi, xs3, w1b, w2b)

    gall3 = _sc_combine(y3, pall3)

    TB = 512
    NB = T // TB
    ws1 = shared_w1.astype(jnp.bfloat16)
    ws2 = shared_w2.astype(jnp.bfloat16)
    out = pl.pallas_call(
        _final_kernel,
        grid=(NB,),
        in_specs=[
            pl.BlockSpec((TB, 16, 128), lambda t: (t, 0, 0)),
            pl.BlockSpec((F, C), lambda t: (0, 0)),
            pl.BlockSpec((C, F), lambda t: (0, 0)),
            pl.BlockSpec((TB, 1), lambda t: (t, 0)),
            pl.BlockSpec((TB, 1), lambda t: (t, 0)),
            pl.BlockSpec((TB, 16, 128), lambda t: (t, 0, 0)),
            pl.BlockSpec((TB, 16, 128), lambda t: (t + NB, 0, 0)),
        ],
        out_specs=pl.BlockSpec((TB, C), lambda t: (t, 0)),
        out_shape=jax.ShapeDtypeStruct((T, C), jnp.float32),
    )(xb3, ws1, ws2, wa, wb, gall3, gall3)

    return out.reshape(B, T, C)


# speedup vs baseline: 2.6241x; 1.1092x over previous
import functools

import jax
import jax.numpy as jnp
from jax import lax
from jax.experimental import pallas as pl
from jax.experimental.pallas import tpu as pltpu
from jax.experimental.pallas import tpu_sc as plsc

E = 8
TOPK = 2
NEG_INF = -1e30

NC = 2
NS = 16
NW = NC * NS
CHUNK = 32

TR = 128


def _router_kernel(x_ref, rw_ref, xb_ref, wa_ref, wb_ref, pa_ref, pb_ref,
                   off_ref):
    x = x_ref[...]
    T = x.shape[0]
    xb_ref[...] = pltpu.bitcast(
        x.reshape(T, 16, 128).astype(jnp.bfloat16), jnp.int32)
    logits = jax.lax.dot_general(
        x.astype(jnp.bfloat16), rw_ref[...].astype(jnp.bfloat16),
        (((1,), (1,)), ((), ())),
        preferred_element_type=jnp.float32)
    lane8 = lax.broadcasted_iota(jnp.int32, (T, E), 1)
    m1 = jnp.max(logits, axis=1, keepdims=True)
    a1 = jnp.argmax(logits, axis=1, keepdims=True)
    masked = jnp.where(lane8 == a1, NEG_INF, logits)
    m2 = jnp.max(masked, axis=1, keepdims=True)
    a2 = jnp.argmax(masked, axis=1, keepdims=True)
    ex = jnp.exp(logits - m1)
    denom = jnp.sum(ex, axis=1, keepdims=True)
    wa_ref[...] = 1.0 / denom
    wb_ref[...] = jnp.exp(m2 - m1) / denom

    na = (lane8 == a1).astype(jnp.bfloat16)
    nb = (lane8 == a2).astype(jnp.bfloat16)
    n = na + nb
    rowi = lax.broadcasted_iota(jnp.int32, (T, T), 0)
    coli = lax.broadcasted_iota(jnp.int32, (T, T), 1)
    tri = jnp.where(coli < rowi, 1.0, 0.0).astype(jnp.bfloat16)
    cx = jax.lax.dot_general(
        tri, n, (((1,), (0,)), ((), ())),
        preferred_element_type=jnp.float32)
    tot = jnp.sum(n.astype(jnp.float32), axis=0, keepdims=True)
    e8r = lax.broadcasted_iota(jnp.int32, (E, E), 0)
    e8c = lax.broadcasted_iota(jnp.int32, (E, E), 1)
    tri8 = jnp.where(e8r < e8c, 1.0, 0.0)
    off = jax.lax.dot_general(
        tot, tri8, (((1,), (0,)), ((), ())),
        preferred_element_type=jnp.float32,
        precision=jax.lax.Precision.HIGHEST)
    dest = cx + off
    pa = jnp.sum(jnp.where(lane8 == a1, dest, 0.0), axis=1, keepdims=True)
    pb = jnp.sum(jnp.where(lane8 == a2, dest, 0.0), axis=1, keepdims=True)
    pa_ref[...] = pa.astype(jnp.int32)
    pb_ref[...] = pb.astype(jnp.int32)
    off_ref[...] = off.astype(jnp.int32)


def _sc_dispatch(xw, pall3):
    T = xw.shape[0]
    S = TOPK * T
    per_w = S // NW
    n_chunks = per_w // CHUNK
    mesh = plsc.VectorSubcoreMesh(core_axis_name="c", subcore_axis_name="s")

    @functools.partial(
        pl.kernel, mesh=mesh,
        out_type=jax.ShapeDtypeStruct((S, 8, 128), jnp.int32),
        scratch_types=[
            pltpu.VMEM((n_chunks, CHUNK), jnp.int32),
            pltpu.VMEM((CHUNK, 8, 128), jnp.int32),
            pltpu.SemaphoreType.DMA,
        ],
    )
    def k(xw_hbm, idx_hbm, xs_hbm, idx_v, rows_v, sem):
        wid = lax.axis_index("s") * NC + lax.axis_index("c")
        pltpu.sync_copy(idx_hbm.at[wid], idx_v)
        tok_base = (wid % NS) * per_w
        for c in range(n_chunks):
            pltpu.sync_copy(xw_hbm.at[pl.ds(tok_base + c * CHUNK, CHUNK)],
                            rows_v)
            pltpu.async_copy(rows_v, xs_hbm.at[idx_v.at[c]], sem).wait()

    return k(xw, pall3)


def _sc_combine(yw, pall3):
    S = yw.shape[0]
    per_w = S // NW
    n_chunks = per_w // CHUNK
    mesh = plsc.VectorSubcoreMesh(core_axis_name="c", subcore_axis_name="s")

    @functools.partial(
        pl.kernel, mesh=mesh,
        out_type=jax.ShapeDtypeStruct((S, 8, 128), jnp.int32),
        scratch_types=[
            pltpu.VMEM((n_chunks, CHUNK), jnp.int32),
            pltpu.VMEM((CHUNK, 8, 128), jnp.int32),
            pltpu.SemaphoreType.DMA,
        ],
    )
    def k(yw_hbm, idx_hbm, g_hbm, idx_v, rows_v, sem):
        wid = lax.axis_index("s") * NC + lax.axis_index("c")
        pltpu.sync_copy(idx_hbm.at[wid], idx_v)
        base = wid * per_w
        for c in range(n_chunks):
            pltpu.async_copy(yw_hbm.at[idx_v.at[c]], rows_v, sem).wait()
            pltpu.sync_copy(rows_v, g_hbm.at[pl.ds(base + c * CHUNK, CHUNK)])

    return k(yw, pall3)


def _gmm_kernel(tid_ref, gid_ref, lo_ref, hi_ref, xs_ref, w1_ref, w2_ref,
                y_ref):
    s = pl.program_id(0)
    lo = lo_ref[s]
    hi = hi_ref[s]

    @pl.when(lo < hi)
    def _():
        xs = pltpu.bitcast(xs_ref[...], jnp.bfloat16).reshape(TR, 2048)
        h = jax.lax.dot_general(
            xs, w1_ref[...], (((1,), (1,)), ((), ())),
            preferred_element_type=jnp.float32)
        h = jnp.maximum(h, 0.0)
        h = h * h
        y = jax.lax.dot_general(
            h.astype(jnp.bfloat16), w2_ref[...], (((1,), (1,)), ((), ())),
            preferred_element_type=jnp.float32)
        rows = tid_ref[s] * TR + lax.broadcasted_iota(jnp.int32,
                                                      (TR, 1, 1), 0)
        m = (rows >= lo) & (rows < hi)
        y3 = pltpu.bitcast(
            y.reshape(TR, 16, 128).astype(jnp.bfloat16), jnp.int32)
        y_ref[...] = jnp.where(m, y3, y_ref[...])


def _final_kernel(xb_ref, ws1_ref, ws2_ref, wa_ref, wb_ref, ga_ref, gb_ref,
                  o_ref):
    xb3 = pltpu.bitcast(xb_ref[...], jnp.bfloat16)
    xb = xb3.reshape(xb3.shape[0], 2048)
    h = jax.lax.dot_general(
        xb, ws1_ref[...], (((1,), (1,)), ((), ())),
        preferred_element_type=jnp.float32)
    h = jnp.maximum(h, 0.0)
    h = h * h
    sh = jax.lax.dot_general(
        h.astype(jnp.bfloat16), ws2_ref[...], (((1,), (1,)), ((), ())),
        preferred_element_type=jnp.float32)
    tb = sh.shape[0]
    ga = pltpu.bitcast(ga_ref[...],
                       jnp.bfloat16).reshape(tb, 2048).astype(jnp.float32)
    gb = pltpu.bitcast(gb_ref[...],
                       jnp.bfloat16).reshape(tb, 2048).astype(jnp.float32)
    o_ref[...] = sh + wa_ref[...] * ga + wb_ref[...] * gb


def kernel(hidden_tensor, router_w, w1_stack, w2_stack, shared_w1, shared_w2):
    B, T, C = hidden_tensor.shape
    F = w1_stack.shape[1]
    S = TOPK * T
    W = C // 2
    x = hidden_tensor.reshape(T, C)

    xb3, wa, wb, pa, pb, off8 = pl.pallas_call(
        _router_kernel,
        out_shape=(
            jax.ShapeDtypeStruct((T, 8, 128), jnp.int32),
            jax.ShapeDtypeStruct((T, 1), jnp.float32),
            jax.ShapeDtypeStruct((T, 1), jnp.float32),
            jax.ShapeDtypeStruct((T, 1), jnp.int32),
            jax.ShapeDtypeStruct((T, 1), jnp.int32),
            jax.ShapeDtypeStruct((1, E), jnp.int32),
        ),
    )(x, router_w)

    pall = jnp.concatenate([pa[:, 0], pb[:, 0]])
    pall3 = pall.reshape(NW, (S // NW) // CHUNK, CHUNK)

    xs3 = _sc_dispatch(xb3, pall3)

    NT = S // TR
    STEPS = NT + E
    off = jnp.concatenate([off8.reshape(E),
                           jnp.array([S], dtype=jnp.int32)])
    start, end = off[:-1], off[1:]
    size = end - start
    t_first = start // TR
    t_last = jnp.where(size > 0, (end - 1) // TR, t_first)
    num_t = jnp.where(size > 0, t_last - t_first + 1, 0)
    sbase = jnp.concatenate([jnp.zeros((1,), jnp.int32),
                             jnp.cumsum(num_t)]).astype(jnp.int32)
    total = sbase[E]
    sarr = jnp.arange(STEPS, dtype=jnp.int32)
    gos = jnp.minimum(
        jnp.sum((sbase[1:][None, :] <= sarr[:, None]).astype(jnp.int32),
                axis=1), E - 1)
    tid = t_first[gos] + (sarr - sbase[:-1][gos])
    valid = sarr < total
    lo = start[gos]
    hi = jnp.where(valid, end[gos], lo)
    tid = jnp.where(valid, tid, NT - 1).astype(jnp.int32)
    gid = gos.astype(jnp.int32)

    w1b = w1_stack.astype(jnp.bfloat16)
    w2b = w2_stack.astype(jnp.bfloat16)
    grid_spec = pltpu.PrefetchScalarGridSpec(
        num_scalar_prefetch=4,
        grid=(STEPS,),
        in_specs=[
            pl.BlockSpec((TR, 8, 128),
                         lambda s, tid, gid, lo, hi: (tid[s], 0, 0)),
            pl.BlockSpec((None, F, C),
                         lambda s, tid, gid, lo, hi: (gid[s], 0, 0)),
            pl.BlockSpec((None, C, F),
                         lambda s, tid, gid, lo, hi: (gid[s], 0, 0)),
        ],
        out_specs=pl.BlockSpec((TR, 8, 128),
                               lambda s, tid, gid, lo, hi: (tid[s], 0, 0)),
    )
    y3 = pl.pallas_call(
        _gmm_kernel,
        grid_spec=grid_spec,
        out_shape=jax.ShapeDtypeStruct((S, 8, 128), jnp.int32),
    )(tid, gid, lo, hi, xs3, w1b, w2b)

    gall3 = _sc_combine(y3, pall3)

    TB = 512
    NB = T // TB
    ws1 = shared_w1.astype(jnp.bfloat16)
    ws2 = shared_w2.astype(jnp.bfloat16)
    out = pl.pallas_call(
        _final_kernel,
        grid=(NB,),
        in_specs=[
            pl.BlockSpec((TB, 8, 128), lambda t: (t, 0, 0)),
            pl.BlockSpec((F, C), lambda t: (0, 0)),
            pl.BlockSpec((C, F), lambda t: (0, 0)),
            pl.BlockSpec((TB, 1), lambda t: (t, 0)),
            pl.BlockSpec((TB, 1), lambda t: (t, 0)),
            pl.BlockSpec((TB, 8, 128), lambda t: (t, 0, 0)),
            pl.BlockSpec((TB, 8, 128), lambda t: (t + NB, 0, 0)),
        ],
        out_specs=pl.BlockSpec((TB, C), lambda t: (t, 0)),
        out_shape=jax.ShapeDtypeStruct((T, C), jnp.float32),
    )(xb3, ws1, ws2, wa, wb, gall3, gall3)

    return out.reshape(B, T, C)
